# Initial kernel scaffold; baseline (speedup 1.0000x reference)
import jax
import jax.numpy as jnp
from jax.experimental import pallas as pl


def kernel(lr, ref, index_map):
    raise NotImplementedError("write your pallas kernel here")


# speedup vs baseline: 9.8008x; 9.8008x over previous
import functools

import jax
import jax.numpy as jnp
from jax import lax
from jax.experimental import pallas as pl
from jax.experimental.pallas import tpu as pltpu
from jax.experimental.pallas import tpu_sc as plsc

H = 512
HI = 510
ZROW = H * H
NW = 32
ROWS_PER_W = H // NW


def _take16(v, lane):
    return lax.gather(
        v, lane[:, None],
        dimension_numbers=lax.GatherDimensionNumbers(
            offset_dims=(), collapsed_slice_dims=(0,), start_index_map=(0,)),
        slice_sizes=(1,),
        mode=lax.GatherScatterMode.PROMISE_IN_BOUNDS)


def _sc_warp(table, imap_pad):
    mesh = plsc.VectorSubcoreMesh(core_axis_name="c", subcore_axis_name="s")

    def body(table_hbm, imap_hbm, out_hbm, m_v, b_v, idx_v, g_v, acc_v, gsem):
        wid = lax.axis_index("s") * 2 + lax.axis_index("c")
        y0 = wid * ROWS_PER_W

        pltpu.sync_copy(imap_hbm.at[pl.ds(y0, 24)], m_v)

        zrow = jnp.full((16,), ZROW, jnp.int32)
        zero = jnp.zeros((16,), jnp.int32)
        hi = jnp.full((16,), HI, jnp.int32)

        def dec_row(r, _):
            def dec_col(c, _):
                v = m_v[r, pl.ds(c * 16, 16)]
                q = lax.div(v, hi)
                b_v[r, pl.ds(c * 16, 16)] = jnp.where(v < zero, zrow, v + q + q)
                return 0
            return lax.fori_loop(0, 33, dec_col, 0)
        lax.fori_loop(0, 18, dec_row, 0)

        def row_body(dy, _):
            for t in range(9):
                ki, kj = t // 3, t % 3
                off = ki * H + kj
                row = dy + 2 - ki

                offv = jnp.full((16,), off, jnp.int32)
                sh = 2 - kj
                iota = lax.iota(jnp.int32, 16)
                lane = jnp.where(iota + sh < 16, iota + sh, iota + (sh - 16))
                hi_m = iota + sh >= 16

                def bld(c, _, t=t, offv=offv, sh=sh, lane=lane, hi_m=hi_m):
                    v0 = b_v[row, pl.ds(c * 16, 16)]
                    if sh == 0:
                        vb = v0
                    else:
                        v1 = b_v[row, pl.ds(c * 16 + 16, 16)]
                        vb = jnp.where(
                            hi_m, _take16(v1, lane), _take16(v0, lane))
                    q = lax.shift_right_logical(c, 3)
                    idx_v[t, q, pl.ds((c & 7) * 16, 16)] = vb + offv
                    return 0
                lax.fori_loop(0, 32, bld, 0)

            copies = [
                pltpu.async_copy(table_hbm.at[idx_v.at[t, q]], g_v.at[t, q], gsem)
                for t in range(9) for q in range(4)
            ]
            for cp in copies:
                cp.wait()

            for q in range(4):
                def sum_body(p, _, q=q):
                    s01 = g_v[0, q, p, :] + g_v[1, q, p, :]
                    s23 = g_v[2, q, p, :] + g_v[3, q, p, :]
                    s45 = g_v[4, q, p, :] + g_v[5, q, p, :]
                    s67 = g_v[6, q, p, :] + g_v[7, q, p, :]
                    acc_v[q * 128 + p, :] = (
                        ((s01 + s23) + (s45 + s67)) + g_v[8, q, p, :])
                    return 0
                lax.fori_loop(0, 128, sum_body, 0)

            pltpu.sync_copy(acc_v, out_hbm.at[pl.ds((y0 + dy) * H, H)])
            return 0
        lax.fori_loop(0, ROWS_PER_W, row_body, 0)

    fn = pl.kernel(
        body,
        out_type=jax.ShapeDtypeStruct((H * H, 16), jnp.float32),
        mesh=mesh,
        scratch_types=[
            pltpu.VMEM((24, 528), jnp.int32),
            pltpu.VMEM((18, 528), jnp.int32),
            pltpu.VMEM((9, 4, 128), jnp.int32),
            pltpu.VMEM((9, 4, 128, 16), jnp.float32),
            pltpu.VMEM((H, 16), jnp.float32),
            pltpu.SemaphoreType.DMA,
        ],
        compiler_params=pltpu.CompilerParams(use_tc_tiling_on_sc=False),
    )
    return fn(table, imap_pad)


@jax.jit
def kernel(lr, ref, index_map):
    del lr
    padded = jnp.pad(ref, ((0, 0), (0, 0), (1, 1), (1, 1)), mode='reflect')
    table = padded[0].transpose(1, 2, 0).reshape(H * H, 16)
    table = jnp.concatenate(
        [table, jnp.zeros((1032, 16), table.dtype)], axis=0)
    m2 = index_map.reshape(HI, HI).astype(jnp.int32)
    imp = jnp.full((520, 528), -1, jnp.int32)
    imp = lax.dynamic_update_slice(imp, m2, (2, 2))
    out = _sc_warp(table, imp)
    return out.reshape(H, H, 16).transpose(2, 0, 1)[None]
